# baseline (speedup 1.0000x reference)
import jax
import jax.numpy as jnp
from jax.experimental import pallas as pl
from jax.experimental.pallas import tpu as pltpu

BATCH = 16384
TILE = 1024
NT = BATCH // TILE
EPS = 1e-5
EMB_DIMS = (6, 6, 3, 6, 6, 3, 2)


def _scale_shift(stats_ref, g_ref, be_ref):
    s = stats_ref[0:1, :]
    ss = stats_ref[1:2, :]
    m = s * (1.0 / BATCH)
    v = ss * (1.0 / BATCH) - m * m
    scale = g_ref[...] * jax.lax.rsqrt(v + EPS)
    shift = be_ref[...] - m * scale
    return scale, shift


def _bn_prelu(x, stats_ref, g_ref, be_ref, a_ref):
    scale, shift = _scale_shift(stats_ref, g_ref, be_ref)
    xn = x.astype(jnp.float32) * scale + shift
    return jnp.where(xn >= 0, xn, a_ref[0, 0] * xn)


def _dot16(x, w_ref):
    return jnp.dot(x.astype(jnp.bfloat16), w_ref[...],
                   preferred_element_type=jnp.float32)


def _accum_stats(first, y, stats_ref):
    @pl.when(first)
    def _():
        stats_ref[...] = jnp.zeros_like(stats_ref)

    srow = jnp.sum(y, axis=0, keepdims=True)
    qrow = jnp.sum(y * y, axis=0, keepdims=True)
    stats_ref[...] += jnp.concatenate([srow, qrow], axis=0)


def _mega_kernel(xn_ref, xe_ref, w1n_ref, m_ref, b1_ref,
                 g1_ref, be1_ref, a1_ref, w2_ref, b2_ref,
                 g2_ref, be2_ref, a2_ref, w3_ref, b3_ref,
                 g3_ref, be3_ref, a3_ref, w4_ref, b4_ref,
                 g4_ref, be4_ref, a4_ref, w5_ref, b5_ref,
                 g5_ref, be5_ref, a5_ref, w6_ref, b6_ref,
                 g6_ref, be6_ref, a6_ref, w7_ref, b7_ref,
                 out_ref,
                 sa, sb, sc, sd, st1, st2, st3, st4, st5, st6):
    p = pl.program_id(0)
    i = pl.program_id(1)
    rows = pl.ds(i * TILE, TILE)

    @pl.when(p == 0)
    def _l1():
        idx = xe_ref[...]
        oh = jnp.concatenate(
            [(idx == 0).astype(jnp.bfloat16), (idx == 1).astype(jnp.bfloat16),
             (idx == 2).astype(jnp.bfloat16),
             jnp.zeros((3, TILE), dtype=jnp.bfloat16)], axis=0)
        y_e = jax.lax.dot_general(oh, m_ref[...], (((0,), (0,)), ((), ())),
                                  preferred_element_type=jnp.float32)
        y = _dot16(xn_ref[...], w1n_ref) + y_e + b1_ref[...]
        sa[rows, :] = y.astype(jnp.bfloat16)
        _accum_stats(i == 0, y, st1)

    @pl.when(p == 1)
    def _l2():
        xn = _bn_prelu(sa[rows, :], st1, g1_ref, be1_ref, a1_ref)
        y = _dot16(xn, w2_ref) + b2_ref[...]
        sb[rows, :] = y.astype(jnp.bfloat16)
        _accum_stats(i == 0, y, st2)

    @pl.when(p == 2)
    def _l3():
        xn = _bn_prelu(sb[rows, :], st2, g2_ref, be2_ref, a2_ref)
        y = _dot16(xn, w3_ref) + b3_ref[...]
        sc[rows, :] = y.astype(jnp.bfloat16)
        _accum_stats(i == 0, y, st3)

    @pl.when(p == 3)
    def _l4():
        xn = _bn_prelu(sc[rows, :], st3, g3_ref, be3_ref, a3_ref)
        y = _dot16(xn, w4_ref) + b4_ref[...]
        sd[rows, :] = y.astype(jnp.bfloat16)
        _accum_stats(i == 0, y, st4)

    @pl.when(p == 4)
    def _l5():
        xn = _bn_prelu(sd[rows, :], st4, g4_ref, be4_ref, a4_ref)
        y = _dot16(xn, w5_ref) + b5_ref[...]
        sb[rows, 0:128] = y.astype(jnp.bfloat16)
        _accum_stats(i == 0, y, st5)

    @pl.when(p == 5)
    def _l6():
        xn = _bn_prelu(sb[rows, 0:128], st5, g5_ref, be5_ref, a5_ref)
        y = _dot16(xn, w6_ref) + b6_ref[...]
        sa[rows, 0:32] = y.astype(jnp.bfloat16)
        _accum_stats(i == 0, y, st6)

    @pl.when(p == 6)
    def _l7():
        xn = _bn_prelu(sa[rows, 0:32], st6, g6_ref, be6_ref, a6_ref)
        y = jnp.dot(xn, w7_ref[...], preferred_element_type=jnp.float32)
        out_ref[...] = jax.nn.sigmoid(y + b7_ref[...])


def _const_spec(shape):
    return pl.BlockSpec(shape, lambda p, i: (0, 0))


@jax.jit
def kernel(X_num, X_embed, E0, E1, E2, E3, E4, E5, E6,
           W1, W2, W3, W4, W5, W6, W7,
           b1, b2, b3, b4, b5, b6, b7,
           g1, g2, g3, g4, g5, g6,
           be1, be2, be3, be4, be5, be6,
           a1, a2, a3, a4, a5, a6):
    wts = [w.T.astype(jnp.bfloat16) for w in (W1, W2, W3, W4, W5, W6)]
    wts.append(W7.T)
    brs = [b.reshape(1, -1) for b in (b1, b2, b3, b4, b5, b6, b7)]
    grs = [g.reshape(1, -1) for g in (g1, g2, g3, g4, g5, g6)]
    bers = [b.reshape(1, -1) for b in (be1, be2, be3, be4, be5, be6)]
    ars = [a.reshape(1, 1) for a in (a1, a2, a3, a4, a5, a6)]
    Es = (E0, E1, E2, E3, E4, E5, E6)

    w1n = wts[0][:30, :]
    mrows = []
    for v in range(3):
        off = 30
        for t, dt in enumerate(EMB_DIMS):
            mrows.append(Es[t][v] @ wts[0][off:off + dt, :].astype(jnp.float32))
            off += dt
    m = jnp.concatenate(
        [jnp.stack(mrows), jnp.zeros((3, wts[0].shape[1]), jnp.float32)],
        axis=0).astype(jnp.bfloat16)

    out = pl.pallas_call(
        _mega_kernel,
        grid=(7, NT),
        in_specs=[
            pl.BlockSpec((TILE, 30), lambda p, i: (jnp.where(p == 0, i, 0), 0)),
            pl.BlockSpec((7, TILE), lambda p, i: (0, jnp.where(p == 0, i, 0))),
            _const_spec(w1n.shape), _const_spec(m.shape), _const_spec((1, 64)),
            _const_spec((1, 64)), _const_spec((1, 64)), _const_spec((1, 1)),
            _const_spec((64, 256)), _const_spec((1, 256)),
            _const_spec((1, 256)), _const_spec((1, 256)), _const_spec((1, 1)),
            _const_spec((256, 512)), _const_spec((1, 512)),
            _const_spec((1, 512)), _const_spec((1, 512)), _const_spec((1, 1)),
            _const_spec((512, 512)), _const_spec((1, 512)),
            _const_spec((1, 512)), _const_spec((1, 512)), _const_spec((1, 1)),
            _const_spec((512, 128)), _const_spec((1, 128)),
            _const_spec((1, 128)), _const_spec((1, 128)), _const_spec((1, 1)),
            _const_spec((128, 32)), _const_spec((1, 32)),
            _const_spec((1, 32)), _const_spec((1, 32)), _const_spec((1, 1)),
            _const_spec((32, 1)), _const_spec((1, 1)),
        ],
        out_specs=pl.BlockSpec((TILE, 1),
                               lambda p, i: (jnp.where(p == 6, i, 0), 0)),
        out_shape=jax.ShapeDtypeStruct((BATCH, 1), jnp.float32),
        scratch_shapes=[
            pltpu.VMEM((BATCH, 64), jnp.bfloat16),
            pltpu.VMEM((BATCH, 256), jnp.bfloat16),
            pltpu.VMEM((BATCH, 512), jnp.bfloat16),
            pltpu.VMEM((BATCH, 512), jnp.bfloat16),
            pltpu.VMEM((2, 64), jnp.float32),
            pltpu.VMEM((2, 256), jnp.float32),
            pltpu.VMEM((2, 512), jnp.float32),
            pltpu.VMEM((2, 512), jnp.float32),
            pltpu.VMEM((2, 128), jnp.float32),
            pltpu.VMEM((2, 32), jnp.float32),
        ],
    )(X_num, X_embed, w1n, m, brs[0],
      grs[0], bers[0], ars[0], wts[1], brs[1],
      grs[1], bers[1], ars[1], wts[2], brs[2],
      grs[2], bers[2], ars[2], wts[3], brs[3],
      grs[3], bers[3], ars[3], wts[4], brs[4],
      grs[4], bers[4], ars[4], wts[5], brs[5],
      grs[5], bers[5], ars[5], wts[6], brs[6])

    return out


# speedup vs baseline: 1.1829x; 1.1829x over previous
import jax
import jax.numpy as jnp
from jax.experimental import pallas as pl
from jax.experimental.pallas import tpu as pltpu

BATCH = 16384
TILE = 2048
NT = BATCH // TILE
EPS = 1e-5
EMB_DIMS = (6, 6, 3, 6, 6, 3, 2)


def _scale_shift(stats_ref, g_ref, be_ref):
    s = stats_ref[0:1, :]
    ss = stats_ref[1:2, :]
    m = s * (1.0 / BATCH)
    v = ss * (1.0 / BATCH) - m * m
    scale = g_ref[...] * jax.lax.rsqrt(v + EPS)
    shift = be_ref[...] - m * scale
    return scale, shift


def _bn_prelu(x, stats_ref, g_ref, be_ref, a_ref):
    scale, shift = _scale_shift(stats_ref, g_ref, be_ref)
    xn = x.astype(jnp.float32) * scale + shift
    return jnp.where(xn >= 0, xn, a_ref[0, 0] * xn)


def _dot16(x, w_ref):
    return jnp.dot(x.astype(jnp.bfloat16), w_ref[...],
                   preferred_element_type=jnp.float32)


def _accum_stats(first, y, stats_ref):
    @pl.when(first)
    def _():
        stats_ref[...] = jnp.zeros_like(stats_ref)

    srow = jnp.sum(y, axis=0, keepdims=True)
    qrow = jnp.sum(y * y, axis=0, keepdims=True)
    stats_ref[...] += jnp.concatenate([srow, qrow], axis=0)


def _mega_kernel(xn_ref, xe_ref, w1n_ref, m_ref, b1_ref,
                 g1_ref, be1_ref, a1_ref, w2_ref, b2_ref,
                 g2_ref, be2_ref, a2_ref, w3_ref, b3_ref,
                 g3_ref, be3_ref, a3_ref, w4_ref, b4_ref,
                 g4_ref, be4_ref, a4_ref, w5_ref, b5_ref,
                 g5_ref, be5_ref, a5_ref, w6_ref, b6_ref,
                 g6_ref, be6_ref, a6_ref, w7_ref, b7_ref,
                 out_ref,
                 sa, sb, sc, sd, st1, st2, st3, st4, st5, st6):
    p = pl.program_id(0)
    i = pl.program_id(1)
    rows = pl.ds(i * TILE, TILE)

    @pl.when(p == 0)
    def _l1():
        idx = xe_ref[...]
        oh = jnp.concatenate(
            [(idx == 0).astype(jnp.bfloat16), (idx == 1).astype(jnp.bfloat16),
             (idx == 2).astype(jnp.bfloat16),
             jnp.zeros((3, TILE), dtype=jnp.bfloat16)], axis=0)
        y_e = jax.lax.dot_general(oh, m_ref[...], (((0,), (0,)), ((), ())),
                                  preferred_element_type=jnp.float32)
        y = _dot16(xn_ref[...], w1n_ref) + y_e + b1_ref[...]
        sa[rows, :] = y.astype(jnp.bfloat16)
        _accum_stats(i == 0, y, st1)

    @pl.when(p == 1)
    def _l2():
        xn = _bn_prelu(sa[rows, :], st1, g1_ref, be1_ref, a1_ref)
        y = _dot16(xn, w2_ref) + b2_ref[...]
        sb[rows, :] = y.astype(jnp.bfloat16)
        _accum_stats(i == 0, y, st2)

    @pl.when(p == 2)
    def _l3():
        xn = _bn_prelu(sb[rows, :], st2, g2_ref, be2_ref, a2_ref)
        y = _dot16(xn, w3_ref) + b3_ref[...]
        sc[rows, :] = y.astype(jnp.bfloat16)
        _accum_stats(i == 0, y, st3)

    @pl.when(p == 3)
    def _l4():
        xn = _bn_prelu(sc[rows, :], st3, g3_ref, be3_ref, a3_ref)
        y = _dot16(xn, w4_ref) + b4_ref[...]
        sd[rows, :] = y.astype(jnp.bfloat16)
        _accum_stats(i == 0, y, st4)

    @pl.when(p == 4)
    def _l5():
        xn = _bn_prelu(sd[rows, :], st4, g4_ref, be4_ref, a4_ref)
        y = _dot16(xn, w5_ref) + b5_ref[...]
        sb[rows, 0:128] = y.astype(jnp.bfloat16)
        _accum_stats(i == 0, y, st5)

    @pl.when(p == 5)
    def _l6():
        xn = _bn_prelu(sb[rows, 0:128], st5, g5_ref, be5_ref, a5_ref)
        y = _dot16(xn, w6_ref) + b6_ref[...]
        sa[rows, 0:32] = y.astype(jnp.bfloat16)
        _accum_stats(i == 0, y, st6)

    @pl.when(p == 6)
    def _l7():
        xn = _bn_prelu(sa[rows, 0:32], st6, g6_ref, be6_ref, a6_ref)
        y = jnp.dot(xn, w7_ref[...], preferred_element_type=jnp.float32)
        out_ref[...] = jax.nn.sigmoid(y + b7_ref[...])


def _const_spec(shape):
    return pl.BlockSpec(shape, lambda p, i: (0, 0))


@jax.jit
def kernel(X_num, X_embed, E0, E1, E2, E3, E4, E5, E6,
           W1, W2, W3, W4, W5, W6, W7,
           b1, b2, b3, b4, b5, b6, b7,
           g1, g2, g3, g4, g5, g6,
           be1, be2, be3, be4, be5, be6,
           a1, a2, a3, a4, a5, a6):
    wts = [w.T.astype(jnp.bfloat16) for w in (W1, W2, W3, W4, W5, W6)]
    wts.append(W7.T)
    brs = [b.reshape(1, -1) for b in (b1, b2, b3, b4, b5, b6, b7)]
    grs = [g.reshape(1, -1) for g in (g1, g2, g3, g4, g5, g6)]
    bers = [b.reshape(1, -1) for b in (be1, be2, be3, be4, be5, be6)]
    ars = [a.reshape(1, 1) for a in (a1, a2, a3, a4, a5, a6)]
    Es = (E0, E1, E2, E3, E4, E5, E6)

    w1n = wts[0][:30, :]
    mrows = []
    for v in range(3):
        off = 30
        for t, dt in enumerate(EMB_DIMS):
            mrows.append(Es[t][v] @ wts[0][off:off + dt, :].astype(jnp.float32))
            off += dt
    m = jnp.concatenate(
        [jnp.stack(mrows), jnp.zeros((3, wts[0].shape[1]), jnp.float32)],
        axis=0).astype(jnp.bfloat16)

    out = pl.pallas_call(
        _mega_kernel,
        grid=(7, NT),
        in_specs=[
            pl.BlockSpec((TILE, 30), lambda p, i: (jnp.where(p == 0, i, 0), 0)),
            pl.BlockSpec((7, TILE), lambda p, i: (0, jnp.where(p == 0, i, 0))),
            _const_spec(w1n.shape), _const_spec(m.shape), _const_spec((1, 64)),
            _const_spec((1, 64)), _const_spec((1, 64)), _const_spec((1, 1)),
            _const_spec((64, 256)), _const_spec((1, 256)),
            _const_spec((1, 256)), _const_spec((1, 256)), _const_spec((1, 1)),
            _const_spec((256, 512)), _const_spec((1, 512)),
            _const_spec((1, 512)), _const_spec((1, 512)), _const_spec((1, 1)),
            _const_spec((512, 512)), _const_spec((1, 512)),
            _const_spec((1, 512)), _const_spec((1, 512)), _const_spec((1, 1)),
            _const_spec((512, 128)), _const_spec((1, 128)),
            _const_spec((1, 128)), _const_spec((1, 128)), _const_spec((1, 1)),
            _const_spec((128, 32)), _const_spec((1, 32)),
            _const_spec((1, 32)), _const_spec((1, 32)), _const_spec((1, 1)),
            _const_spec((32, 1)), _const_spec((1, 1)),
        ],
        out_specs=pl.BlockSpec((TILE, 1),
                               lambda p, i: (jnp.where(p == 6, i, 0), 0)),
        out_shape=jax.ShapeDtypeStruct((BATCH, 1), jnp.float32),
        scratch_shapes=[
            pltpu.VMEM((BATCH, 64), jnp.bfloat16),
            pltpu.VMEM((BATCH, 256), jnp.bfloat16),
            pltpu.VMEM((BATCH, 512), jnp.bfloat16),
            pltpu.VMEM((BATCH, 512), jnp.bfloat16),
            pltpu.VMEM((2, 64), jnp.float32),
            pltpu.VMEM((2, 256), jnp.float32),
            pltpu.VMEM((2, 512), jnp.float32),
            pltpu.VMEM((2, 512), jnp.float32),
            pltpu.VMEM((2, 128), jnp.float32),
            pltpu.VMEM((2, 32), jnp.float32),
        ],
    )(X_num, X_embed, w1n, m, brs[0],
      grs[0], bers[0], ars[0], wts[1], brs[1],
      grs[1], bers[1], ars[1], wts[2], brs[2],
      grs[2], bers[2], ars[2], wts[3], brs[3],
      grs[3], bers[3], ars[3], wts[4], brs[4],
      grs[4], bers[4], ars[4], wts[5], brs[5],
      grs[5], bers[5], ars[5], wts[6], brs[6])

    return out
